# baseline (speedup 1.0000x reference)
import functools

import jax
import jax.numpy as jnp
from jax import lax
from jax.experimental import pallas as pl

TOPK = 8


def _router_body(x1_ref, x2_ref, wt_ref, bias_ref, w_out_ref, i_out_ref, *,
                 n_experts):
    tb = x1_ref.shape[0]
    h2 = x1_ref.shape[1]
    logits = (jnp.dot(x1_ref[...], wt_ref[0:h2],
                      preferred_element_type=jnp.float32) +
              jnp.dot(x2_ref[...], wt_ref[h2:2 * h2],
                      preferred_element_type=jnp.float32))
    scores = jax.nn.sigmoid(logits)
    bal = scores + bias_ref[...]
    iotaf = lax.broadcasted_iota(jnp.int32, (tb, n_experts), 1).astype(
        jnp.float32)
    combo = iotaf + 0.5 * scores

    work = bal
    neg_inf = jnp.float32(-jnp.inf)
    big = jnp.float32(1e9)
    combs = []
    for _ in range(TOPK):
        m = jnp.max(work, axis=1, keepdims=True)
        combined = jnp.min(jnp.where(work == m, combo, big), axis=1,
                           keepdims=True)
        work = jnp.where(combo == combined, neg_inf, work)
        combs.append(combined)

    ccat = jnp.concatenate(combs, axis=1)
    icat = ccat.astype(jnp.int32)
    wcat = (ccat - icat.astype(jnp.float32)) * 2.0
    l1 = jnp.maximum(jnp.sum(jnp.abs(wcat), axis=1, keepdims=True), 1e-12)
    w_out_ref[...] = wcat / l1
    i_out_ref[...] = icat


def kernel(x, W, router_bias):
    tokens, hidden = x.shape
    n_experts = W.shape[0]
    tb = 1024
    grid = (tokens // tb,)
    wt = W.T
    bias2d = router_bias.reshape(1, n_experts)

    body = functools.partial(_router_body, n_experts=n_experts)
    w_out, i_out = pl.pallas_call(
        body,
        grid=grid,
        in_specs=[
            pl.BlockSpec((tb, hidden // 2), lambda i: (i, 0)),
            pl.BlockSpec((tb, hidden // 2), lambda i: (i, 1)),
            pl.BlockSpec((hidden, n_experts), lambda i: (0, 0)),
            pl.BlockSpec((1, n_experts), lambda i: (0, 0)),
        ],
        out_specs=[
            pl.BlockSpec((tb, TOPK), lambda i: (i, 0)),
            pl.BlockSpec((tb, TOPK), lambda i: (i, 0)),
        ],
        out_shape=[
            jax.ShapeDtypeStruct((tokens, TOPK), jnp.float32),
            jax.ShapeDtypeStruct((tokens, TOPK), jnp.int32),
        ],
    )(x, x, wt, bias2d)
    return (w_out, i_out)
